# Initial kernel scaffold; baseline (speedup 1.0000x reference)
import jax
import jax.numpy as jnp
from jax.experimental import pallas as pl


def kernel(x, grid, w1, b1, w2, b2, w3, b3):
    raise NotImplementedError("write your pallas kernel here")


# speedup vs baseline: 2.2889x; 2.2889x over previous
import functools

import jax
import jax.numpy as jnp
from jax import lax
from jax.experimental import pallas as pl
from jax.experimental.pallas import tpu as pltpu
from jax.experimental.pallas import tpu_sc as plsc

L = 16
T = 524288
F = 2
PI2 = 2654435761
PI3 = 805459861
MASK = T - 1
OFFSETS = [(i, j, k) for i in (0, 1) for j in (0, 1) for k in (0, 1)]
CORNER_OFF = [(dx + dy * PI2 + dz * PI3) % (1 << 32) for dx, dy, dz in OFFSETS]

NC = 2
NS = 16
NW = NC * NS
LANES = 16
CH = 512
NB = CH // LANES


def _featurize_kernel(xt, levels, table, out, xv, lv_v, idx_buf, off_buf,
                      val_buf, wgt_buf, ft_buf, sem):
    wid = lax.axis_index("s") * NC + lax.axis_index("c")
    n = out.shape[1]
    ppw = n // NW
    nchunk = ppw // CH
    wb = wid * ppw

    pltpu.sync_copy(levels, lv_v)
    pltpu.sync_copy(xt.at[:, pl.ds(wb, ppw)], xv)

    lane = jnp.arange(LANES, dtype=jnp.int32)
    one_i = jnp.full((LANES,), 1, jnp.int32)

    @pl.loop(0, nchunk)
    def _chunk(ch):
        @pl.loop(0, NB)
        def _batch(b):
            boff = b * LANES
            p0 = ch * CH + boff
            xs = xv[0, pl.ds(p0, LANES)]
            ys = xv[1, pl.ds(p0, LANES)]
            zs = xv[2, pl.ds(p0, LANES)]

            lvv = lv_v[...]
            for l in range(L):
                lv = jnp.full((LANES,), lvv[l], jnp.float32)
                xl = xs * lv
                yl = ys * lv
                zl = zs * lv
                xi = xl.astype(jnp.int32)
                yi = yl.astype(jnp.int32)
                zi = zl.astype(jnp.int32)
                fx = xl - xi.astype(jnp.float32)
                fy = yl - yi.astype(jnp.float32)
                fz = zl - zi.astype(jnp.float32)
                base = (xi.astype(jnp.uint32)
                        + yi.astype(jnp.uint32) * jnp.uint32(PI2)
                        + zi.astype(jnp.uint32) * jnp.uint32(PI3))
                gx0 = 1.0 - fx
                gy0 = 1.0 - fy
                gz0 = 1.0 - fz
                wxy = (gx0 * gy0, gx0 * fy, fx * gy0, fx * fy)
                lrow = jnp.uint32(l * (T // 8))
                for c, (dx, dy, dz) in enumerate(OFFSETS):
                    h = (base + jnp.uint32(CORNER_OFF[c])) & jnp.uint32(MASK)
                    idx_buf[l, pl.ds(c * LANES, LANES)] = (
                        ((h >> 3) | lrow).astype(jnp.int32))
                    off_buf[l, pl.ds(c * LANES, LANES)] = (
                        ((h & jnp.uint32(7)) << 1).astype(jnp.int32))
                    w = wxy[2 * dx + dy] * (fz if dz else gz0)
                    wgt_buf[l, pl.ds(c * LANES, LANES)] = w

            copies = [pltpu.async_copy(
                table.at[idx_buf.at[l]], val_buf.at[l], sem)
                for l in range(L)]
            for d in copies:
                d.wait()

            for l in range(L):
                li = jnp.full((LANES,), l, jnp.int32)
                acc0 = None
                acc1 = None
                for c in range(8):
                    w = wgt_buf[l, pl.ds(c * LANES, LANES)]
                    pos = c * LANES + lane
                    o0 = off_buf[l, pl.ds(c * LANES, LANES)]
                    v0 = plsc.load_gather(val_buf, [li, pos, o0])
                    v1 = plsc.load_gather(val_buf, [li, pos, o0 + one_i])
                    acc0 = w * v0 if acc0 is None else acc0 + w * v0
                    acc1 = w * v1 if acc1 is None else acc1 + w * v1
                ft_buf[2 * l, pl.ds(boff, LANES)] = acc0
                ft_buf[2 * l + 1, pl.ds(boff, LANES)] = acc1

        pltpu.sync_copy(ft_buf, out.at[:, pl.ds(wb + ch * CH, CH)])


def _featurize(xt, levels, table):
    n = xt.shape[1]
    mesh = plsc.VectorSubcoreMesh(core_axis_name="c", subcore_axis_name="s",
                                  num_cores=NC, num_subcores=NS)
    return pl.kernel(
        _featurize_kernel,
        out_type=jax.ShapeDtypeStruct((L * F, n), jnp.float32),
        mesh=mesh,
        scratch_types=[
            pltpu.VMEM((3, n // NW), jnp.float32),
            pltpu.VMEM((L,), jnp.float32),
            pltpu.VMEM((L, 8 * LANES), jnp.int32),
            pltpu.VMEM((L, 8 * LANES), jnp.int32),
            pltpu.VMEM((L, 8 * LANES, 16), jnp.float32),
            pltpu.VMEM((L, 8 * LANES), jnp.float32),
            pltpu.VMEM((L * F, CH), jnp.float32),
            pltpu.SemaphoreType.DMA,
        ],
        compiler_params=pltpu.CompilerParams(needs_layout_passes=False,
                                             use_tc_tiling_on_sc=False),
    )(xt, levels, table)


def _mlp_body(ft_ref, w1_ref, b1_ref, w2_ref, b2_ref, w3_ref, b3_ref,
              sig_ref, alp_ref):
    ft = ft_ref[...]
    dn = (((0,), (0,)), ((), ()))
    h1 = lax.dot_general(w1_ref[...], ft, dn,
                         preferred_element_type=jnp.float32)
    h1 = h1 + jnp.reshape(b1_ref[...], (b1_ref.shape[1], 1))
    h1 = jnp.where(h1 >= 0, h1, 0.01 * h1)
    h2 = lax.dot_general(w2_ref[...], h1, dn,
                         preferred_element_type=jnp.float32)
    h2 = h2 + jnp.reshape(b2_ref[...], (b2_ref.shape[1], 1))
    h2 = jnp.where(h2 >= 0, h2, 0.01 * h2)
    o = lax.dot_general(w3_ref[...], h2, dn,
                        preferred_element_type=jnp.float32)
    o = o + jnp.reshape(b3_ref[...], (b3_ref.shape[1], 1))
    sig = o[0:1, :]
    alp = o[1:2, :]
    sig_ref[...] = jnp.where(sig > -1.0, sig, 0.0)
    alp_ref[...] = jnp.minimum(alp, 0.0) * 0.1


def _mlp(ft, w1, b1, w2, b2, w3, b3):
    n = ft.shape[1]
    bn = 4096
    grid = n // bn
    full = lambda shape: pl.BlockSpec(shape, lambda i: (0, 0))
    return pl.pallas_call(
        _mlp_body,
        grid=(grid,),
        in_specs=[
            pl.BlockSpec((L * F, bn), lambda i: (0, i)),
            full(w1.shape), full((1, b1.shape[0])),
            full(w2.shape), full((1, b2.shape[0])),
            full(w3.shape), full((1, b3.shape[0])),
        ],
        out_specs=[
            pl.BlockSpec((1, bn), lambda i: (0, i)),
            pl.BlockSpec((1, bn), lambda i: (0, i)),
        ],
        out_shape=[
            jax.ShapeDtypeStruct((1, n), jnp.float32),
            jax.ShapeDtypeStruct((1, n), jnp.float32),
        ],
    )(ft, w1, b1.reshape(1, -1), w2, b2.reshape(1, -1), w3,
      b3.reshape(1, -1))


def kernel(x, grid, w1, b1, w2, b2, w3, b3):
    n = x.shape[0]
    xt = x.T
    table = grid.reshape(L * T // 8, 8 * F)
    levels = 2.0 * 2.0 ** (0.5 * jnp.arange(L, dtype=jnp.float32))
    ft = _featurize(xt, levels, table)
    sig, alp = _mlp(ft, w1, b1, w2, b2, w3, b3)
    return (sig.reshape(n), alp.reshape(n), jnp.zeros((n,), jnp.float32))


# speedup vs baseline: 10.1836x; 4.4491x over previous
import functools

import jax
import jax.numpy as jnp
from jax import lax
from jax.experimental import pallas as pl
from jax.experimental.pallas import tpu as pltpu
from jax.experimental.pallas import tpu_sc as plsc

L = 16
T = 524288
F = 2
PI2 = 2654435761
PI3 = 805459861
MASK = T - 1
OFFSETS = [(i, j, k) for i in (0, 1) for j in (0, 1) for k in (0, 1)]
CORNER_OFF = [(dx + dy * PI2 + dz * PI3) % (1 << 32) for dx, dy, dz in OFFSETS]

NC = 2
NS = 16
NW = NC * NS
LANES = 16
CH = 512
NB = CH // LANES


HL = L // 2

NPAIR = L * T // 128
RB = 128


def _repack_kernel(src, rep, buf):
    wid = lax.axis_index("s") * NC + lax.axis_index("c")
    base = wid * (NPAIR // NW)

    @pl.loop(0, NPAIR // NW // RB)
    def _c(i):
        p0 = base + i * RB
        for e in range(F):
            pltpu.sync_copy(src.at[pl.ds(p0, RB), e], buf.at[:, :, e])
        pltpu.sync_copy(buf, rep.at[pl.ds(p0, RB)])


def _repack(src):
    mesh = plsc.VectorSubcoreMesh(core_axis_name="c", subcore_axis_name="s",
                                  num_cores=NC, num_subcores=NS)
    return pl.kernel(
        _repack_kernel,
        out_type=jax.ShapeDtypeStruct((NPAIR, 16, F, 8), jnp.float32),
        mesh=mesh,
        scratch_types=[pltpu.VMEM((RB, 16, F, 8), jnp.float32)],
        compiler_params=pltpu.CompilerParams(needs_layout_passes=False,
                                             use_tc_tiling_on_sc=False),
    )(src)


def _featurize_kernel(xt, levels, table, out, xv, lv_v, idx_buf, off_buf,
                      val_buf, frc_buf, ft_buf, sem_a, sem_b):
    wid = lax.axis_index("s") * NC + lax.axis_index("c")
    n = out.shape[1]
    ppw = n // NW
    nchunk = ppw // CH
    wb = wid * ppw

    pltpu.sync_copy(levels, lv_v)
    pltpu.sync_copy(xt.at[:, pl.ds(wb, ppw)], xv)

    lane = jnp.arange(LANES, dtype=jnp.int32)

    def compute_issue(p0, half, sem):
        xs = xv[0, pl.ds(p0, LANES)]
        ys = xv[1, pl.ds(p0, LANES)]
        zs = xv[2, pl.ds(p0, LANES)]
        lvv = lv_v[...]
        for l in range(half * HL, half * HL + HL):
            lv = jnp.full((LANES,), lvv[l], jnp.float32)
            xl = xs * lv
            yl = ys * lv
            zl = zs * lv
            xi = xl.astype(jnp.int32)
            yi = yl.astype(jnp.int32)
            zi = zl.astype(jnp.int32)
            fx = xl - xi.astype(jnp.float32)
            fy = yl - yi.astype(jnp.float32)
            fz = zl - zi.astype(jnp.float32)
            base = (xi.astype(jnp.uint32)
                    + yi.astype(jnp.uint32) * jnp.uint32(PI2)
                    + zi.astype(jnp.uint32) * jnp.uint32(PI3))
            frc_buf[l, pl.ds(0, LANES)] = fx
            frc_buf[l, pl.ds(LANES, LANES)] = fy
            frc_buf[l, pl.ds(2 * LANES, LANES)] = fz
            lrow = jnp.uint32(l << 16)
            lb = (l - half * HL) * 8 * LANES
            for c in range(8):
                h = (base + jnp.uint32(CORNER_OFF[c])) & jnp.uint32(MASK)
                idx_buf[half, pl.ds(lb + c * LANES, LANES)] = (
                    (lrow | (h >> 3)).astype(jnp.int32))
                off_buf[l, pl.ds(c * LANES, LANES)] = (
                    (h & jnp.uint32(7)).astype(jnp.int32))
        pltpu.async_copy(table.at[idx_buf.at[half]], val_buf.at[half], sem)

    def drain(half, sem):
        pltpu.make_async_copy(
            table.at[idx_buf.at[half]], val_buf.at[half], sem).wait()

    def reduce(boff, half):
        sl = jnp.full((LANES,), half, jnp.int32)
        for l in range(half * HL, half * HL + HL):
            lb = (l - half * HL) * 8 * LANES
            fx = frc_buf[l, pl.ds(0, LANES)]
            fy = frc_buf[l, pl.ds(LANES, LANES)]
            fz = frc_buf[l, pl.ds(2 * LANES, LANES)]
            for f in range(F):
                fo = 8 * f
                v = []
                for c in range(8):
                    o = off_buf[l, pl.ds(c * LANES, LANES)] + fo
                    pos = lb + c * LANES + lane
                    v.append(plsc.load_gather(val_buf, [sl, pos, o]))
                u = [v[2 * i] + fz * (v[2 * i + 1] - v[2 * i])
                     for i in range(4)]
                t0 = u[0] + fy * (u[1] - u[0])
                t1 = u[2] + fy * (u[3] - u[2])
                ft_buf[2 * l + f, pl.ds(boff, LANES)] = (
                    t0 + fx * (t1 - t0))

    @pl.loop(0, nchunk)
    def _chunk(ch):
        cp = ch * CH
        compute_issue(cp, 0, sem_a)
        compute_issue(cp, 1, sem_b)

        @pl.loop(0, NB - 1)
        def _batch(b):
            boff = b * LANES
            nxt = cp + boff + LANES
            drain(0, sem_a)
            reduce(boff, 0)
            compute_issue(nxt, 0, sem_a)
            drain(1, sem_b)
            reduce(boff, 1)
            compute_issue(nxt, 1, sem_b)

        lb = (NB - 1) * LANES
        drain(0, sem_a)
        reduce(lb, 0)
        drain(1, sem_b)
        reduce(lb, 1)
        pltpu.sync_copy(ft_buf, out.at[:, pl.ds(wb + ch * CH, CH)])


def _featurize(xt, levels, table):
    n = xt.shape[1]
    mesh = plsc.VectorSubcoreMesh(core_axis_name="c", subcore_axis_name="s",
                                  num_cores=NC, num_subcores=NS)
    return pl.kernel(
        _featurize_kernel,
        out_type=jax.ShapeDtypeStruct((L * F, n), jnp.float32),
        mesh=mesh,
        scratch_types=[
            pltpu.VMEM((3, n // NW), jnp.float32),
            pltpu.VMEM((L,), jnp.float32),
            pltpu.VMEM((2, HL * 8 * LANES), jnp.int32),
            pltpu.VMEM((L, 8 * LANES), jnp.int32),
            pltpu.VMEM((2, HL * 8 * LANES, 16), jnp.float32),
            pltpu.VMEM((L, 3 * LANES), jnp.float32),
            pltpu.VMEM((L * F, CH), jnp.float32),
            pltpu.SemaphoreType.DMA,
            pltpu.SemaphoreType.DMA,
        ],
        compiler_params=pltpu.CompilerParams(needs_layout_passes=False,
                                             use_tc_tiling_on_sc=False),
    )(xt, levels, table)


def _mlp_body(ft_ref, w1_ref, b1_ref, w2_ref, b2_ref, w3_ref, b3_ref,
              sig_ref, alp_ref):
    ft = ft_ref[...]
    dn = (((0,), (0,)), ((), ()))
    h1 = lax.dot_general(w1_ref[...], ft, dn,
                         preferred_element_type=jnp.float32)
    h1 = h1 + jnp.reshape(b1_ref[...], (b1_ref.shape[1], 1))
    h1 = jnp.where(h1 >= 0, h1, 0.01 * h1)
    h2 = lax.dot_general(w2_ref[...], h1, dn,
                         preferred_element_type=jnp.float32)
    h2 = h2 + jnp.reshape(b2_ref[...], (b2_ref.shape[1], 1))
    h2 = jnp.where(h2 >= 0, h2, 0.01 * h2)
    o = lax.dot_general(w3_ref[...], h2, dn,
                        preferred_element_type=jnp.float32)
    o = o + jnp.reshape(b3_ref[...], (b3_ref.shape[1], 1))
    sig = o[0:1, :]
    alp = o[1:2, :]
    sig_ref[...] = jnp.where(sig > -1.0, sig, 0.0)
    alp_ref[...] = jnp.minimum(alp, 0.0) * 0.1


def _mlp(ft, w1, b1, w2, b2, w3, b3):
    n = ft.shape[1]
    bn = 4096
    grid = n // bn
    full = lambda shape: pl.BlockSpec(shape, lambda i: (0, 0))
    return pl.pallas_call(
        _mlp_body,
        grid=(grid,),
        in_specs=[
            pl.BlockSpec((L * F, bn), lambda i: (0, i)),
            full(w1.shape), full((1, b1.shape[0])),
            full(w2.shape), full((1, b2.shape[0])),
            full(w3.shape), full((1, b3.shape[0])),
        ],
        out_specs=[
            pl.BlockSpec((1, bn), lambda i: (0, i)),
            pl.BlockSpec((1, bn), lambda i: (0, i)),
        ],
        out_shape=[
            jax.ShapeDtypeStruct((1, n), jnp.float32),
            jax.ShapeDtypeStruct((1, n), jnp.float32),
        ],
    )(ft, w1, b1.reshape(1, -1), w2, b2.reshape(1, -1), w3,
      b3.reshape(1, -1))


def kernel(x, grid, w1, b1, w2, b2, w3, b3):
    n = x.shape[0]
    xt = x.T
    src = (grid.reshape(L, T // 128, 128, F)
           .transpose(0, 1, 3, 2)
           .reshape(NPAIR, F, 16, 8))
    rep = _repack(src)
    levels = 2.0 * 2.0 ** (0.5 * jnp.arange(L, dtype=jnp.float32))
    ft = _featurize(xt, levels, rep.reshape(L * T // 8, 16))
    sig, alp = _mlp(ft, w1, b1, w2, b2, w3, b3)
    return (sig.reshape(n), alp.reshape(n), jnp.zeros((n,), jnp.float32))


# speedup vs baseline: 14.6367x; 1.4373x over previous
import functools

import jax
import jax.numpy as jnp
from jax import lax
from jax.experimental import pallas as pl
from jax.experimental.pallas import tpu as pltpu
from jax.experimental.pallas import tpu_sc as plsc

L = 16
T = 524288
F = 2
PI2 = 2654435761
PI3 = 805459861
MASK = T - 1
OFFSETS = [(i, j, k) for i in (0, 1) for j in (0, 1) for k in (0, 1)]
CORNER_OFF = [(dx + dy * PI2 + dz * PI3) % (1 << 32) for dx, dy, dz in OFFSETS]

NC = 2
NS = 16
NW = NC * NS
LANES = 16
CH = 512
NB = CH // LANES


HL = L // 2
SLOT = HL * F * 8 * LANES


def _featurize_kernel(xt, levels, table, out, xv, lv_v, idx_buf,
                      val_buf, frc_buf, ft_buf, sem_a, sem_b):
    wid = lax.axis_index("s") * NC + lax.axis_index("c")
    n = out.shape[1]
    ppw = n // NW
    nchunk = ppw // CH
    wb = wid * ppw

    pltpu.sync_copy(levels, lv_v)
    pltpu.sync_copy(xt.at[:, pl.ds(wb, ppw)], xv)

    def compute_issue(p0, half, sem):
        xs = xv[0, pl.ds(p0, LANES)]
        ys = xv[1, pl.ds(p0, LANES)]
        zs = xv[2, pl.ds(p0, LANES)]
        lvv = lv_v[...]
        for l in range(half * HL, half * HL + HL):
            ll = l - half * HL
            lv = jnp.full((LANES,), lvv[l], jnp.float32)
            xl = xs * lv
            yl = ys * lv
            zl = zs * lv
            xi = xl.astype(jnp.int32)
            yi = yl.astype(jnp.int32)
            zi = zl.astype(jnp.int32)
            fx = xl - xi.astype(jnp.float32)
            fy = yl - yi.astype(jnp.float32)
            fz = zl - zi.astype(jnp.float32)
            base = (xi.astype(jnp.uint32)
                    + yi.astype(jnp.uint32) * jnp.uint32(PI2)
                    + zi.astype(jnp.uint32) * jnp.uint32(PI3))
            frc_buf[l, pl.ds(0, LANES)] = fx
            frc_buf[l, pl.ds(LANES, LANES)] = fy
            frc_buf[l, pl.ds(2 * LANES, LANES)] = fz
            for c in range(8):
                h = (base + jnp.uint32(CORNER_OFF[c])) & jnp.uint32(MASK)
                e0 = ((jnp.uint32(l << 20))
                      | ((h >> 7) << 8) | (h & jnp.uint32(127)))
                idx_buf[half, pl.ds(ll * 256 + c * LANES, LANES)] = (
                    e0.astype(jnp.int32))
                idx_buf[half, pl.ds(ll * 256 + 128 + c * LANES, LANES)] = (
                    (e0 | jnp.uint32(128)).astype(jnp.int32))
        pltpu.async_copy(table.at[idx_buf.at[half]], val_buf.at[half], sem)

    def drain(half, sem):
        pltpu.make_async_copy(
            table.at[idx_buf.at[half]], val_buf.at[half], sem).wait()

    def reduce(boff, half):
        for l in range(half * HL, half * HL + HL):
            ll = l - half * HL
            fx = frc_buf[l, pl.ds(0, LANES)]
            fy = frc_buf[l, pl.ds(LANES, LANES)]
            fz = frc_buf[l, pl.ds(2 * LANES, LANES)]
            for f in range(F):
                vb = ll * 256 + f * 128
                v = [val_buf[half, pl.ds(vb + c * LANES, LANES)]
                     for c in range(8)]
                u = [v[2 * i] + fz * (v[2 * i + 1] - v[2 * i])
                     for i in range(4)]
                t0 = u[0] + fy * (u[1] - u[0])
                t1 = u[2] + fy * (u[3] - u[2])
                ft_buf[2 * l + f, pl.ds(boff, LANES)] = (
                    t0 + fx * (t1 - t0))

    @pl.loop(0, nchunk)
    def _chunk(ch):
        cp = ch * CH
        compute_issue(cp, 0, sem_a)
        compute_issue(cp, 1, sem_b)

        @pl.loop(0, NB - 1)
        def _batch(b):
            boff = b * LANES
            nxt = cp + boff + LANES
            drain(0, sem_a)
            reduce(boff, 0)
            compute_issue(nxt, 0, sem_a)
            drain(1, sem_b)
            reduce(boff, 1)
            compute_issue(nxt, 1, sem_b)

        lb = (NB - 1) * LANES
        drain(0, sem_a)
        reduce(lb, 0)
        drain(1, sem_b)
        reduce(lb, 1)
        pltpu.sync_copy(ft_buf, out.at[:, pl.ds(wb + ch * CH, CH)])


def _featurize(xt, levels, table):
    n = xt.shape[1]
    mesh = plsc.VectorSubcoreMesh(core_axis_name="c", subcore_axis_name="s",
                                  num_cores=NC, num_subcores=NS)
    return pl.kernel(
        _featurize_kernel,
        out_type=jax.ShapeDtypeStruct((L * F, n), jnp.float32),
        mesh=mesh,
        scratch_types=[
            pltpu.VMEM((3, n // NW), jnp.float32),
            pltpu.VMEM((L,), jnp.float32),
            pltpu.VMEM((2, SLOT), jnp.int32),
            pltpu.VMEM((2, SLOT), jnp.float32),
            pltpu.VMEM((L, 3 * LANES), jnp.float32),
            pltpu.VMEM((L * F, CH), jnp.float32),
            pltpu.SemaphoreType.DMA,
            pltpu.SemaphoreType.DMA,
        ],
        compiler_params=pltpu.CompilerParams(needs_layout_passes=False,
                                             use_tc_tiling_on_sc=False),
    )(xt, levels, table)


def _mlp_body(ft_ref, w1_ref, b1_ref, w2_ref, b2_ref, w3_ref, b3_ref,
              sig_ref, alp_ref):
    ft = ft_ref[...]
    dn = (((0,), (0,)), ((), ()))
    h1 = lax.dot_general(w1_ref[...], ft, dn,
                         preferred_element_type=jnp.float32)
    h1 = h1 + jnp.reshape(b1_ref[...], (b1_ref.shape[1], 1))
    h1 = jnp.where(h1 >= 0, h1, 0.01 * h1)
    h2 = lax.dot_general(w2_ref[...], h1, dn,
                         preferred_element_type=jnp.float32)
    h2 = h2 + jnp.reshape(b2_ref[...], (b2_ref.shape[1], 1))
    h2 = jnp.where(h2 >= 0, h2, 0.01 * h2)
    o = lax.dot_general(w3_ref[...], h2, dn,
                        preferred_element_type=jnp.float32)
    o = o + jnp.reshape(b3_ref[...], (b3_ref.shape[1], 1))
    sig = o[0:1, :]
    alp = o[1:2, :]
    sig_ref[...] = jnp.where(sig > -1.0, sig, 0.0)
    alp_ref[...] = jnp.minimum(alp, 0.0) * 0.1


def _mlp(ft, w1, b1, w2, b2, w3, b3):
    n = ft.shape[1]
    bn = 4096
    grid = n // bn
    full = lambda shape: pl.BlockSpec(shape, lambda i: (0, 0))
    return pl.pallas_call(
        _mlp_body,
        grid=(grid,),
        in_specs=[
            pl.BlockSpec((L * F, bn), lambda i: (0, i)),
            full(w1.shape), full((1, b1.shape[0])),
            full(w2.shape), full((1, b2.shape[0])),
            full(w3.shape), full((1, b3.shape[0])),
        ],
        out_specs=[
            pl.BlockSpec((1, bn), lambda i: (0, i)),
            pl.BlockSpec((1, bn), lambda i: (0, i)),
        ],
        out_shape=[
            jax.ShapeDtypeStruct((1, n), jnp.float32),
            jax.ShapeDtypeStruct((1, n), jnp.float32),
        ],
    )(ft, w1, b1.reshape(1, -1), w2, b2.reshape(1, -1), w3,
      b3.reshape(1, -1))


def kernel(x, grid, w1, b1, w2, b2, w3, b3):
    n = x.shape[0]
    xt = x.T
    flat = (grid.reshape(L, T // 128, 128, F)
            .transpose(0, 1, 3, 2)
            .reshape(L * T * F))
    levels = 2.0 * 2.0 ** (0.5 * jnp.arange(L, dtype=jnp.float32))
    ft = _featurize(xt, levels, flat)
    sig, alp = _mlp(ft, w1, b1, w2, b2, w3, b3)
    return (sig.reshape(n), alp.reshape(n), jnp.zeros((n,), jnp.float32))
